# Initial kernel scaffold; baseline (speedup 1.0000x reference)
import functools

import jax
import jax.numpy as jnp
from jax import lax
from jax.experimental import pallas as pl
from jax.experimental.pallas import tpu as pltpu
from jax.experimental.pallas import tpu_sc as plsc

N = 10000
E = 320000
D = 128

NC = 2
NS = 16
NW = NC * NS

EPT = E // NW
CH = 128
NFULL = EPT // CH
REM = EPT - NFULL * CH
RPT = N // NS

_mesh = plsc.VectorSubcoreMesh(core_axis_name="c", subcore_axis_name="s")


@functools.partial(
    pl.kernel,
    out_type=jax.ShapeDtypeStruct((NC, N, D), jnp.float32),
    mesh=_mesh,
    scratch_types=[
        pltpu.VMEM_SHARED((N, D), jnp.float32),
        pltpu.VMEM((CH, D), jnp.float32),
        pltpu.VMEM((CH,), jnp.int32),
        pltpu.VMEM((CH,), jnp.int32),
        pltpu.VMEM((REM, D), jnp.float32),
        pltpu.VMEM((REM,), jnp.int32),
        pltpu.VMEM((REM,), jnp.int32),
        pltpu.SemaphoreType.DMA,
    ],
)
def _spmm_sc(x_hbm, src_hbm, dst_hbm, zeros_hbm, out_hbm,
             acc, rows, sidx, didx, rrows, rsidx, rdidx, sem):
    c = lax.axis_index("c")
    s = lax.axis_index("s")
    w = s * NC + c

    r0 = s * RPT
    pltpu.sync_copy(zeros_hbm.at[pl.ds(r0, RPT)], acc.at[pl.ds(r0, RPT)])
    plsc.subcore_barrier()

    e0 = w * EPT

    def body(i, _):
        base = e0 + i * CH
        pltpu.sync_copy(src_hbm.at[pl.ds(base, CH)], sidx)
        pltpu.sync_copy(dst_hbm.at[pl.ds(base, CH)], didx)
        pltpu.async_copy(x_hbm.at[sidx], rows, sem).wait()
        pltpu.sync_copy(rows, acc.at[didx], add=True)
        return 0

    lax.fori_loop(0, NFULL, body, 0)

    base = e0 + NFULL * CH
    pltpu.sync_copy(src_hbm.at[pl.ds(base, REM)], rsidx)
    pltpu.sync_copy(dst_hbm.at[pl.ds(base, REM)], rdidx)
    pltpu.async_copy(x_hbm.at[rsidx], rrows, sem).wait()
    pltpu.sync_copy(rrows, acc.at[rdidx], add=True)

    plsc.subcore_barrier()
    pltpu.sync_copy(acc.at[pl.ds(r0, RPT)], out_hbm.at[c, pl.ds(r0, RPT)])


BR = 1000


def _dense_body(p0_ref, p1_ref, wt_ref, b_ref, a_ref, o_ref):
    neigh = p0_ref[...] + p1_ref[...]
    h = jnp.dot(neigh, wt_ref[...], preferred_element_type=jnp.float32)
    h = h + b_ref[...]
    a = a_ref[0, 0]
    h = jnp.where(h >= 0, h, a * h)
    m = jnp.max(h, axis=1, keepdims=True)
    e = jnp.exp(h - m)
    o_ref[...] = e / jnp.sum(e, axis=1, keepdims=True)


def kernel(x, edge_index, W, b, prelu_a):
    ei = edge_index.astype(jnp.int32)
    src = ei[0]
    dst = ei[1]
    zeros = jnp.zeros((N, D), jnp.float32)

    parts = _spmm_sc(x, src, dst, zeros)

    wt = W.T
    b2 = b.reshape(1, D)
    a2 = prelu_a.reshape(1, 1)

    out = pl.pallas_call(
        _dense_body,
        grid=(N // BR,),
        in_specs=[
            pl.BlockSpec((BR, D), lambda i: (i, 0)),
            pl.BlockSpec((BR, D), lambda i: (i, 0)),
            pl.BlockSpec((D, D), lambda i: (0, 0)),
            pl.BlockSpec((1, D), lambda i: (0, 0)),
            pl.BlockSpec(memory_space=pltpu.SMEM),
        ],
        out_specs=pl.BlockSpec((BR, D), lambda i: (i, 0)),
        out_shape=jax.ShapeDtypeStruct((N, D), jnp.float32),
    )(parts[0], parts[1], wt, b2, a2)
    return out


# speedup vs baseline: 6.7651x; 6.7651x over previous
import functools

import jax
import jax.numpy as jnp
from jax import lax
from jax.experimental import pallas as pl
from jax.experimental.pallas import tpu as pltpu
from jax.experimental.pallas import tpu_sc as plsc

N = 10000
E = 320000
D = 128

NC = 2
NS = 16
NW = NC * NS

EPT = E // NW
CH = 128
NFULL = EPT // CH
REM = EPT - NFULL * CH
NP = 10240
RPT = NP // NS

_mesh = plsc.VectorSubcoreMesh(core_axis_name="c", subcore_axis_name="s")


@functools.partial(
    pl.kernel,
    out_type=jax.ShapeDtypeStruct((NC, NP, D), jnp.float32),
    mesh=_mesh,
    scratch_types=[
        pltpu.VMEM_SHARED((NP, D), jnp.float32),
        pltpu.VMEM((CH, D), jnp.float32),
        pltpu.VMEM((CH,), jnp.int32),
        pltpu.VMEM((CH,), jnp.int32),
        pltpu.VMEM((REM, D), jnp.float32),
        pltpu.VMEM((REM,), jnp.int32),
        pltpu.VMEM((REM,), jnp.int32),
        pltpu.SemaphoreType.DMA,
    ],
)
def _spmm_sc(x_hbm, src_hbm, dst_hbm, zeros_hbm, out_hbm,
             acc, rows, sidx, didx, rrows, rsidx, rdidx, sem):
    c = lax.axis_index("c")
    s = lax.axis_index("s")
    w = s * NC + c

    r0 = s * RPT
    pltpu.sync_copy(zeros_hbm.at[pl.ds(r0, RPT)], acc.at[pl.ds(r0, RPT)])
    plsc.subcore_barrier()

    e0 = w * EPT

    def body(i, _):
        base = e0 + i * CH
        pltpu.sync_copy(src_hbm.at[pl.ds(base, CH)], sidx)
        pltpu.sync_copy(dst_hbm.at[pl.ds(base, CH)], didx)
        pltpu.async_copy(x_hbm.at[sidx], rows, sem).wait()
        pltpu.sync_copy(rows, acc.at[didx], add=True)
        return 0

    lax.fori_loop(0, NFULL, body, 0)

    base = e0 + NFULL * CH
    pltpu.sync_copy(src_hbm.at[pl.ds(base, REM)], rsidx)
    pltpu.sync_copy(dst_hbm.at[pl.ds(base, REM)], rdidx)
    pltpu.async_copy(x_hbm.at[rsidx], rrows, sem).wait()
    pltpu.sync_copy(rrows, acc.at[rdidx], add=True)

    plsc.subcore_barrier()
    pltpu.sync_copy(acc.at[pl.ds(r0, RPT)], out_hbm.at[c, pl.ds(r0, RPT)])


BR = 1000


def _dense_body(p_ref, wt_ref, b_ref, a_ref, o_ref):
    neigh = p_ref[0] + p_ref[1]
    h = jnp.dot(neigh, wt_ref[...], preferred_element_type=jnp.float32)
    h = h + b_ref[...]
    a = a_ref[0, 0]
    h = jnp.where(h >= 0, h, a * h)
    m = jnp.max(h, axis=1, keepdims=True)
    e = jnp.exp(h - m)
    o_ref[...] = e / jnp.sum(e, axis=1, keepdims=True)


def kernel(x, edge_index, W, b, prelu_a):
    ei = edge_index.astype(jnp.int32)
    src = ei[0]
    dst = ei[1]
    zeros = jnp.zeros((NP, D), jnp.float32)

    parts = _spmm_sc(x, src, dst, zeros)

    wt = W.T
    b2 = b.reshape(1, D)
    a2 = prelu_a.reshape(1, 1)

    out = pl.pallas_call(
        _dense_body,
        grid=(N // BR,),
        in_specs=[
            pl.BlockSpec((NC, BR, D), lambda i: (0, i, 0)),
            pl.BlockSpec((D, D), lambda i: (0, 0)),
            pl.BlockSpec((1, D), lambda i: (0, 0)),
            pl.BlockSpec(memory_space=pltpu.SMEM),
        ],
        out_specs=pl.BlockSpec((BR, D), lambda i: (i, 0)),
        out_shape=jax.ShapeDtypeStruct((N, D), jnp.float32),
    )(parts, wt, b2, a2)
    return out
